# baseline (speedup 1.0000x reference)
import functools

import jax
import jax.numpy as jnp
from jax import lax
from jax.experimental import pallas as pl
from jax.experimental.pallas import tpu as pltpu
from jax.experimental.pallas import tpu_sc as plsc

N = 10000
E = 320000
D = 128

NC = 2
NS = 16
NW = NC * NS
EW = E // NW
C = 40
NBUF = 4
NGRP = EW // C
IB = 25
NIB = NGRP // IB

NPAD = 10240
RT = NPAD // NS
WC = 40


def _sc_body(x_hbm, src_hbm, dst_hbm, out_hbm,
             src_v0, dst_v0, src_v1, dst_v1, rows0, rows1, rows2, rows3,
             gsem0, gsem1, gsem2, gsem3, ssem0, ssem1, ssem2, ssem3, isem,
             acc):
    c = lax.axis_index("c")
    s = lax.axis_index("s")
    wid = s * NC + c
    rows = (rows0, rows1, rows2, rows3)
    gsem = (gsem0, gsem1, gsem2, gsem3)
    ssem = (ssem0, ssem1, ssem2, ssem3)
    idx = ((src_v0, dst_v0), (src_v1, dst_v1))

    tbase = s * RT

    def fire_gather(src_v, k, slot):
        pltpu.async_copy(x_hbm.at[src_v.at[k]], rows[slot], gsem[slot])

    def drain_gather(src_v, k, slot):
        pltpu.make_async_copy(
            x_hbm.at[src_v.at[k]], rows[slot], gsem[slot]).wait()

    def fire_scatter(dst_v, k, slot):
        pltpu.async_copy(rows[slot], acc.at[dst_v.at[k]], ssem[slot],
                         add=True)

    def drain_scatter(dst_v, k, slot):
        pltpu.make_async_copy(
            rows[slot], acc.at[dst_v.at[k]], ssem[slot]).wait()

    def stage_idx(b, par):
        return (pltpu.make_async_copy(src_hbm.at[wid, b], idx[par][0], isem),
                pltpu.make_async_copy(dst_hbm.at[wid, b], idx[par][1], isem))

    pltpu.sync_copy(src_hbm.at[wid, 0], src_v0)
    pltpu.sync_copy(dst_hbm.at[wid, 0], dst_v0)
    for k in range(NBUF - 1):
        fire_gather(src_v0, k, k)

    @functools.partial(lax.fori_loop, 0, C * 8, init_val=None)
    def _(t, _):
        rows3[t // 8, pl.ds((t % 8) * 16, 16)] = jnp.zeros((16,), jnp.float32)
        return None

    @functools.partial(lax.fori_loop, 0, RT // WC, init_val=None)
    def _(r, _):
        pltpu.sync_copy(rows3, acc.at[pl.ds(tbase + r * WC, WC)])
        return None

    plsc.subcore_barrier()

    def run_block(b, par):
        src_v, dst_v = idx[par]

        @pl.when(b >= 1)
        def _():
            for d in stage_idx(b, par):
                d.wait()
            for k in range(NBUF - 1):
                fire_gather(src_v, k, k)

        @pl.when(b + 1 < NIB)
        def _():
            for d in stage_idx(b + 1, 1 - par):
                d.start()

        def step(j, p):
            @pl.when(j >= 1)
            def _():
                drain_scatter(dst_v, j - 1, (p + NBUF - 1) % NBUF)

            @pl.when(j + NBUF - 1 < IB)
            def _():
                fire_gather(src_v, j + NBUF - 1, (p + NBUF - 1) % NBUF)

            drain_gather(src_v, j, p)
            fire_scatter(dst_v, j, p)

        @functools.partial(lax.fori_loop, 0, IB, init_val=None)
        def _(j, _):
            for p in range(NBUF):
                @pl.when(j % NBUF == p)
                def _(p=p):
                    step(j, p)

            return None

        drain_scatter(dst_v, IB - 1, (IB - 1) % NBUF)

    @functools.partial(lax.fori_loop, 0, NIB, init_val=None)
    def _(b, _):
        @pl.when(b % 2 == 0)
        def _():
            run_block(b, 0)

        @pl.when(b % 2 == 1)
        def _():
            run_block(b, 1)

        return None

    plsc.subcore_barrier()

    def wr_copy(r):
        return pltpu.make_async_copy(
            acc.at[pl.ds(tbase + r * WC, WC)],
            out_hbm.at[c].at[pl.ds(tbase + r * WC, WC)], gsem0)

    @functools.partial(lax.fori_loop, 0, RT // WC, init_val=None)
    def _(r, _):
        wr_copy(r).start()
        return None

    @functools.partial(lax.fori_loop, 0, RT // WC, init_val=None)
    def _(r, _):
        wr_copy(r).wait()
        return None


_sc_segment_sum = functools.partial(
    pl.kernel,
    out_type=jax.ShapeDtypeStruct((NC, NPAD, D), jnp.float32),
    mesh=plsc.VectorSubcoreMesh(
        core_axis_name="c", subcore_axis_name="s",
        num_cores=NC, num_subcores=NS),
    scratch_types=(
        [pltpu.VMEM((IB, C), jnp.int32)] * 4
        + [pltpu.VMEM((C, D), jnp.float32)] * NBUF
        + [pltpu.SemaphoreType.DMA] * (2 * NBUF + 1)
        + [pltpu.VMEM_SHARED((NPAD, D), jnp.float32)]
    ),
)(_sc_body)


BLK = 1000


def _mlp_body(relu_out, h_ref, p_ref, wa_ref, ba_ref, wb_ref, bb_ref, o_ref):
    z = h_ref[...] + p_ref[0] + p_ref[1]
    a = jnp.dot(z, wa_ref[...], preferred_element_type=jnp.float32) + ba_ref[...]
    a = jnp.where(a > 0, a, a * 0.01)
    o = jnp.dot(a, wb_ref[...], preferred_element_type=jnp.float32) + bb_ref[...]
    if relu_out:
        o = jnp.where(o > 0, o, o * 0.01)
    o_ref[...] = o


def _mlp_tc(h, p, wa_t, ba, wb_t, bb, relu_out):
    row_spec = pl.BlockSpec((BLK, D), lambda i: (i, 0))
    part_spec = pl.BlockSpec((2, BLK, D), lambda i: (0, i, 0))
    full_spec = pl.BlockSpec((D, D), lambda i: (0, 0))
    bias_spec = pl.BlockSpec((1, D), lambda i: (0, 0))
    return pl.pallas_call(
        functools.partial(_mlp_body, relu_out),
        grid=(N // BLK,),
        in_specs=[row_spec, part_spec,
                  full_spec, bias_spec, full_spec, bias_spec],
        out_specs=row_spec,
        out_shape=jax.ShapeDtypeStruct((N, D), jnp.float32),
    )(h, p, wa_t, ba.reshape(1, D), wb_t, bb.reshape(1, D))


def kernel(x, edge_index, W1a, b1a, W1b, b1b, W2a, b2a, W2b, b2b,
           W3a, b3a, W3b, b3b):
    src = edge_index[0].reshape(NW, NIB, IB, C)
    dst = edge_index[1].reshape(NW, NIB, IB, C)

    w3b_t = jnp.zeros((D, D), jnp.float32).at[:, :2].set(W3b.T)
    b3b_p = jnp.zeros((D,), jnp.float32).at[:2].set(b3b)

    p = _sc_segment_sum(x, src, dst)
    h = _mlp_tc(x, p, W1a.T, b1a, W1b.T, b1b, relu_out=True)

    p = _sc_segment_sum(h, src, dst)
    h = _mlp_tc(h, p, W2a.T, b2a, W2b.T, b2b, relu_out=True)

    p = _sc_segment_sum(h, src, dst)
    out = _mlp_tc(h, p, W3a.T, b3a, w3b_t, b3b_p, relu_out=False)

    return out[:, :2]


# speedup vs baseline: 1.0254x; 1.0254x over previous
import functools

import jax
import jax.numpy as jnp
from jax import lax
from jax.experimental import pallas as pl
from jax.experimental.pallas import tpu as pltpu
from jax.experimental.pallas import tpu_sc as plsc

N = 10000
E = 320000
D = 128

NC = 2
NS = 16
NW = NC * NS
EW = E // NW
C = 25
NBUF = 5
NGRP = EW // C
IB = 25
NIB = NGRP // IB

NPAD = 10240
RT = NPAD // NS
WC = 16


def _sc_body(x_hbm, src_hbm, dst_hbm, out_hbm,
             src_v0, dst_v0, src_v1, dst_v1,
             rows0, rows1, rows2, rows3, rows4,
             gsem0, gsem1, gsem2, gsem3, gsem4,
             ssem0, ssem1, ssem2, ssem3, ssem4, isem,
             acc):
    c = lax.axis_index("c")
    s = lax.axis_index("s")
    wid = s * NC + c
    rows = (rows0, rows1, rows2, rows3, rows4)
    gsem = (gsem0, gsem1, gsem2, gsem3, gsem4)
    ssem = (ssem0, ssem1, ssem2, ssem3, ssem4)
    idx = ((src_v0, dst_v0), (src_v1, dst_v1))

    tbase = s * RT

    def fire_gather(src_v, k, slot):
        pltpu.async_copy(x_hbm.at[src_v.at[k]], rows[slot], gsem[slot])

    def drain_gather(src_v, k, slot):
        pltpu.make_async_copy(
            x_hbm.at[src_v.at[k]], rows[slot], gsem[slot]).wait()

    def fire_scatter(dst_v, k, slot):
        pltpu.async_copy(rows[slot], acc.at[dst_v.at[k]], ssem[slot],
                         add=True)

    def drain_scatter(dst_v, k, slot):
        pltpu.make_async_copy(
            rows[slot], acc.at[dst_v.at[k]], ssem[slot]).wait()

    def stage_idx(b, par):
        return (pltpu.make_async_copy(src_hbm.at[wid, b], idx[par][0], isem),
                pltpu.make_async_copy(dst_hbm.at[wid, b], idx[par][1], isem))

    pltpu.sync_copy(src_hbm.at[wid, 0], src_v0)
    pltpu.sync_copy(dst_hbm.at[wid, 0], dst_v0)
    for k in range(NBUF - 1):
        fire_gather(src_v0, k, k)

    @functools.partial(lax.fori_loop, 0, WC * 8, init_val=None)
    def _(t, _):
        rows4[t // 8, pl.ds((t % 8) * 16, 16)] = jnp.zeros((16,), jnp.float32)
        return None

    zsrc = rows4.at[pl.ds(0, WC)]

    def z_copy(r):
        return pltpu.make_async_copy(
            zsrc, acc.at[pl.ds(tbase + r * WC, WC)], ssem0)

    @functools.partial(lax.fori_loop, 0, RT // WC, init_val=None)
    def _(r, _):
        z_copy(r).start()
        return None

    @functools.partial(lax.fori_loop, 0, RT // WC, init_val=None)
    def _(r, _):
        z_copy(r).wait()
        return None

    plsc.subcore_barrier()

    def run_block(b, par):
        src_v, dst_v = idx[par]
        src_vo, dst_vo = idx[1 - par]

        @pl.when(b >= 1)
        def _():
            drain_scatter(dst_vo, IB - 1, NBUF - 1)

        @pl.when(b + 1 < NIB)
        def _():
            for d in stage_idx(b + 1, 1 - par):
                d.start()

        def step(j, p):
            vac = (p + NBUF - 1) % NBUF

            @pl.when(j >= 1)
            def _():
                drain_scatter(dst_v, j - 1, vac)

            @pl.when(jnp.logical_and(j == IB - NBUF, b + 1 < NIB))
            def _():
                for d in stage_idx(b + 1, 1 - par):
                    d.wait()

            @pl.when(j + NBUF - 1 < IB)
            def _():
                fire_gather(src_v, j + NBUF - 1, vac)

            @pl.when(jnp.logical_and(j + NBUF - 1 >= IB, b + 1 < NIB))
            def _():
                fire_gather(src_vo, j + NBUF - 1 - IB, vac)

            drain_gather(src_v, j, p)
            fire_scatter(dst_v, j, p)

        @functools.partial(lax.fori_loop, 0, IB, init_val=None)
        def _(j, _):
            for p in range(NBUF):
                @pl.when(j % NBUF == p)
                def _(p=p):
                    step(j, p)

            return None

    @functools.partial(lax.fori_loop, 0, NIB, init_val=None)
    def _(b, _):
        @pl.when(b % 2 == 0)
        def _():
            run_block(b, 0)

        @pl.when(b % 2 == 1)
        def _():
            run_block(b, 1)

        return None

    drain_scatter(idx[(NIB - 1) % 2][1], IB - 1, NBUF - 1)

    plsc.subcore_barrier()

    def wr_copy(r):
        return pltpu.make_async_copy(
            acc.at[pl.ds(tbase + r * WC, WC)],
            out_hbm.at[c].at[pl.ds(tbase + r * WC, WC)], gsem0)

    @functools.partial(lax.fori_loop, 0, RT // WC, init_val=None)
    def _(r, _):
        wr_copy(r).start()
        return None

    @functools.partial(lax.fori_loop, 0, RT // WC, init_val=None)
    def _(r, _):
        wr_copy(r).wait()
        return None


_sc_segment_sum = functools.partial(
    pl.kernel,
    out_type=jax.ShapeDtypeStruct((NC, NPAD, D), jnp.float32),
    mesh=plsc.VectorSubcoreMesh(
        core_axis_name="c", subcore_axis_name="s",
        num_cores=NC, num_subcores=NS),
    scratch_types=(
        [pltpu.VMEM((IB, C), jnp.int32)] * 4
        + [pltpu.VMEM((C, D), jnp.float32)] * NBUF
        + [pltpu.SemaphoreType.DMA] * (2 * NBUF + 1)
        + [pltpu.VMEM_SHARED((NPAD, D), jnp.float32)]
    ),
)(_sc_body)


BLK = 1000


def _mlp_body(relu_out, h_ref, p_ref, wa_ref, ba_ref, wb_ref, bb_ref, o_ref):
    z = h_ref[...] + p_ref[0] + p_ref[1]
    a = jnp.dot(z, wa_ref[...], preferred_element_type=jnp.float32) + ba_ref[...]
    a = jnp.where(a > 0, a, a * 0.01)
    o = jnp.dot(a, wb_ref[...], preferred_element_type=jnp.float32) + bb_ref[...]
    if relu_out:
        o = jnp.where(o > 0, o, o * 0.01)
    o_ref[...] = o


def _mlp_tc(h, p, wa_t, ba, wb_t, bb, relu_out):
    row_spec = pl.BlockSpec((BLK, D), lambda i: (i, 0))
    part_spec = pl.BlockSpec((2, BLK, D), lambda i: (0, i, 0))
    full_spec = pl.BlockSpec((D, D), lambda i: (0, 0))
    bias_spec = pl.BlockSpec((1, D), lambda i: (0, 0))
    return pl.pallas_call(
        functools.partial(_mlp_body, relu_out),
        grid=(N // BLK,),
        in_specs=[row_spec, part_spec,
                  full_spec, bias_spec, full_spec, bias_spec],
        out_specs=row_spec,
        out_shape=jax.ShapeDtypeStruct((N, D), jnp.float32),
    )(h, p, wa_t, ba.reshape(1, D), wb_t, bb.reshape(1, D))


def kernel(x, edge_index, W1a, b1a, W1b, b1b, W2a, b2a, W2b, b2b,
           W3a, b3a, W3b, b3b):
    src = edge_index[0].reshape(NW, NIB, IB, C)
    dst = edge_index[1].reshape(NW, NIB, IB, C)

    w3b_t = jnp.zeros((D, D), jnp.float32).at[:, :2].set(W3b.T)
    b3b_p = jnp.zeros((D,), jnp.float32).at[:2].set(b3b)

    p = _sc_segment_sum(x, src, dst)
    h = _mlp_tc(x, p, W1a.T, b1a, W1b.T, b1b, relu_out=True)

    p = _sc_segment_sum(h, src, dst)
    h = _mlp_tc(h, p, W2a.T, b2a, W2b.T, b2b, relu_out=True)

    p = _sc_segment_sum(h, src, dst)
    out = _mlp_tc(h, p, W3a.T, b3a, w3b_t, b3b_p, relu_out=False)

    return out[:, :2]


# speedup vs baseline: 1.0477x; 1.0217x over previous
import functools

import jax
import jax.numpy as jnp
from jax import lax
from jax.experimental import pallas as pl
from jax.experimental.pallas import tpu as pltpu
from jax.experimental.pallas import tpu_sc as plsc

N = 10000
E = 320000
D = 128

NC = 2
NS = 16
NW = NC * NS
EW = E // NW
C = 25
NBUF = 5
NGRP = EW // C
IB = 25
NIB = NGRP // IB

NPAD = 10240
RT = NPAD // NS
WC = 16


def _sc_body(x_hbm, src_hbm, dst_hbm, out_hbm,
             src_v0, dst_v0, src_v1, dst_v1,
             rows0, rows1, rows2, rows3, rows4,
             gsem0, gsem1, gsem2, gsem3, gsem4,
             ssem0, ssem1, ssem2, ssem3, ssem4, isem,
             acc):
    c = lax.axis_index("c")
    s = lax.axis_index("s")
    wid = s * NC + c
    rows = (rows0, rows1, rows2, rows3, rows4)
    gsem = (gsem0, gsem1, gsem2, gsem3, gsem4)
    ssem = (ssem0, ssem1, ssem2, ssem3, ssem4)
    idx = ((src_v0, dst_v0), (src_v1, dst_v1))

    tbase = s * RT

    def fire_gather(src_v, k, slot):
        pltpu.async_copy(x_hbm.at[src_v.at[k]], rows[slot], gsem[slot])

    def drain_gather(src_v, k, slot):
        pltpu.make_async_copy(
            x_hbm.at[src_v.at[k]], rows[slot], gsem[slot]).wait()

    def fire_scatter(dst_v, k, slot):
        pltpu.async_copy(rows[slot], acc.at[dst_v.at[k]], ssem[slot],
                         add=True)

    def drain_scatter(dst_v, k, slot):
        pltpu.make_async_copy(
            rows[slot], acc.at[dst_v.at[k]], ssem[slot]).wait()

    def stage_idx(b, par):
        return (pltpu.make_async_copy(src_hbm.at[wid, b], idx[par][0], isem),
                pltpu.make_async_copy(dst_hbm.at[wid, b], idx[par][1], isem))

    pltpu.sync_copy(src_hbm.at[wid, 0], src_v0)
    pltpu.sync_copy(dst_hbm.at[wid, 0], dst_v0)
    for k in range(NBUF - 1):
        fire_gather(src_v0, k, k)

    @functools.partial(lax.fori_loop, 0, WC * 8, init_val=None)
    def _(t, _):
        rows4[t // 8, pl.ds((t % 8) * 16, 16)] = jnp.zeros((16,), jnp.float32)
        return None

    zsrc = rows4.at[pl.ds(0, WC)]

    def z_copy(r):
        return pltpu.make_async_copy(
            zsrc, acc.at[pl.ds(tbase + r * WC, WC)], ssem0)

    @functools.partial(lax.fori_loop, 0, RT // WC, init_val=None)
    def _(r, _):
        z_copy(r).start()
        return None

    @functools.partial(lax.fori_loop, 0, RT // WC, init_val=None)
    def _(r, _):
        z_copy(r).wait()
        return None

    plsc.subcore_barrier()

    def run_block(b, par):
        src_v, dst_v = idx[par]
        src_vo, dst_vo = idx[1 - par]

        @pl.when(b >= 1)
        def _():
            drain_scatter(dst_vo, IB - 1, NBUF - 1)

        @pl.when(b + 1 < NIB)
        def _():
            for d in stage_idx(b + 1, 1 - par):
                d.start()

        def step(j, p):
            vac = (p + NBUF - 1) % NBUF

            @pl.when(j >= 1)
            def _():
                drain_scatter(dst_v, j - 1, vac)

            @pl.when(jnp.logical_and(j == IB - NBUF, b + 1 < NIB))
            def _():
                for d in stage_idx(b + 1, 1 - par):
                    d.wait()

            @pl.when(j + NBUF - 1 < IB)
            def _():
                fire_gather(src_v, j + NBUF - 1, vac)

            @pl.when(jnp.logical_and(j + NBUF - 1 >= IB, b + 1 < NIB))
            def _():
                fire_gather(src_vo, j + NBUF - 1 - IB, vac)

            drain_gather(src_v, j, p)
            fire_scatter(dst_v, j, p)

        @functools.partial(lax.fori_loop, 0, IB, init_val=None)
        def _(j, _):
            for p in range(NBUF):
                @pl.when(j % NBUF == p)
                def _(p=p):
                    step(j, p)

            return None

    @functools.partial(lax.fori_loop, 0, NIB, init_val=None)
    def _(b, _):
        @pl.when(b % 2 == 0)
        def _():
            run_block(b, 0)

        @pl.when(b % 2 == 1)
        def _():
            run_block(b, 1)

        return None

    drain_scatter(idx[(NIB - 1) % 2][1], IB - 1, NBUF - 1)

    plsc.subcore_barrier()

    def wr_copy(r):
        return pltpu.make_async_copy(
            acc.at[pl.ds(tbase + r * WC, WC)],
            out_hbm.at[c].at[pl.ds(tbase + r * WC, WC)], gsem0)

    @functools.partial(lax.fori_loop, 0, RT // WC, init_val=None)
    def _(r, _):
        wr_copy(r).start()
        return None

    @functools.partial(lax.fori_loop, 0, RT // WC, init_val=None)
    def _(r, _):
        wr_copy(r).wait()
        return None


_sc_segment_sum = functools.partial(
    pl.kernel,
    out_type=jax.ShapeDtypeStruct((NC, NPAD, D), jnp.float32),
    mesh=plsc.VectorSubcoreMesh(
        core_axis_name="c", subcore_axis_name="s",
        num_cores=NC, num_subcores=NS),
    scratch_types=(
        [pltpu.VMEM((IB, C), jnp.int32)] * 4
        + [pltpu.VMEM((C, D), jnp.float32)] * NBUF
        + [pltpu.SemaphoreType.DMA] * (2 * NBUF + 1)
        + [pltpu.VMEM_SHARED((NPAD, D), jnp.float32)]
    ),
)(_sc_body)


BLK = 2000


def _mlp_body(relu_out, h_ref, p_ref, wa_ref, ba_ref, wb_ref, bb_ref, o_ref):
    z = h_ref[...] + p_ref[0] + p_ref[1]
    a = jnp.dot(z, wa_ref[...], preferred_element_type=jnp.float32) + ba_ref[...]
    a = jnp.where(a > 0, a, a * 0.01)
    o = jnp.dot(a, wb_ref[...], preferred_element_type=jnp.float32) + bb_ref[...]
    if relu_out:
        o = jnp.where(o > 0, o, o * 0.01)
    o_ref[...] = o


def _mlp_tc(h, p, wa_t, ba, wb_t, bb, relu_out):
    row_spec = pl.BlockSpec((BLK, D), lambda i: (i, 0))
    part_spec = pl.BlockSpec((2, BLK, D), lambda i: (0, i, 0))
    full_spec = pl.BlockSpec((D, D), lambda i: (0, 0))
    bias_spec = pl.BlockSpec((1, D), lambda i: (0, 0))
    return pl.pallas_call(
        functools.partial(_mlp_body, relu_out),
        grid=(N // BLK,),
        in_specs=[row_spec, part_spec,
                  full_spec, bias_spec, full_spec, bias_spec],
        out_specs=row_spec,
        out_shape=jax.ShapeDtypeStruct((N, D), jnp.float32),
    )(h, p, wa_t, ba.reshape(1, D), wb_t, bb.reshape(1, D))


def kernel(x, edge_index, W1a, b1a, W1b, b1b, W2a, b2a, W2b, b2b,
           W3a, b3a, W3b, b3b):
    src = edge_index[0].reshape(NW, NIB, IB, C)
    dst = edge_index[1].reshape(NW, NIB, IB, C)

    w3b_t = jnp.zeros((D, D), jnp.float32).at[:, :2].set(W3b.T)
    b3b_p = jnp.zeros((D,), jnp.float32).at[:2].set(b3b)

    p = _sc_segment_sum(x, src, dst)
    h = _mlp_tc(x, p, W1a.T, b1a, W1b.T, b1b, relu_out=True)

    p = _sc_segment_sum(h, src, dst)
    h = _mlp_tc(h, p, W2a.T, b2a, W2b.T, b2b, relu_out=True)

    p = _sc_segment_sum(h, src, dst)
    out = _mlp_tc(h, p, W3a.T, b3a, w3b_t, b3b_p, relu_out=False)

    return out[:, :2]


# speedup vs baseline: 1.0603x; 1.0121x over previous
import functools

import jax
import jax.numpy as jnp
from jax import lax
from jax.experimental import pallas as pl
from jax.experimental.pallas import tpu as pltpu
from jax.experimental.pallas import tpu_sc as plsc

N = 10000
E = 320000
D = 128

NC = 2
NS = 16
NW = NC * NS
EW = E // NW
C = 25
NBUF = 5
NGRP = EW // C
IB = 25
NIB = NGRP // IB

NPAD = 10240
RT = NPAD // NS
WC = 16


def _sc_body(x_hbm, src_hbm, dst_hbm, out_hbm,
             src_v0, dst_v0, src_v1, dst_v1,
             rows0, rows1, rows2, rows3, rows4,
             gsem0, gsem1, gsem2, gsem3, gsem4,
             ssem0, ssem1, ssem2, ssem3, ssem4, isem,
             acc):
    c = lax.axis_index("c")
    s = lax.axis_index("s")
    wid = s * NC + c
    rows = (rows0, rows1, rows2, rows3, rows4)
    gsem = (gsem0, gsem1, gsem2, gsem3, gsem4)
    ssem = (ssem0, ssem1, ssem2, ssem3, ssem4)
    idx = ((src_v0, dst_v0), (src_v1, dst_v1))

    tbase = s * RT

    def fire_gather(src_v, k, slot):
        pltpu.async_copy(x_hbm.at[src_v.at[k]], rows[slot], gsem[slot])

    def drain_gather(src_v, k, slot):
        pltpu.make_async_copy(
            x_hbm.at[src_v.at[k]], rows[slot], gsem[slot]).wait()

    def fire_scatter(dst_v, k, slot):
        pltpu.async_copy(rows[slot], acc.at[dst_v.at[k]], ssem[slot],
                         add=True)

    def drain_scatter(dst_v, k, slot):
        pltpu.make_async_copy(
            rows[slot], acc.at[dst_v.at[k]], ssem[slot]).wait()

    def stage_idx(b, par):
        return (pltpu.make_async_copy(src_hbm.at[wid, b], idx[par][0], isem),
                pltpu.make_async_copy(dst_hbm.at[wid, b], idx[par][1], isem))

    pltpu.sync_copy(src_hbm.at[wid, 0], src_v0)
    pltpu.sync_copy(dst_hbm.at[wid, 0], dst_v0)
    for k in range(NBUF - 1):
        fire_gather(src_v0, k, k)

    @functools.partial(lax.fori_loop, 0, WC * 8, init_val=None)
    def _(t, _):
        rows4[t // 8, pl.ds((t % 8) * 16, 16)] = jnp.zeros((16,), jnp.float32)
        return None

    zsrc = rows4.at[pl.ds(0, WC)]

    def z_copy(r):
        return pltpu.make_async_copy(
            zsrc, acc.at[pl.ds(tbase + r * WC, WC)], ssem0)

    @functools.partial(lax.fori_loop, 0, RT // WC, init_val=None)
    def _(r, _):
        z_copy(r).start()
        return None

    @functools.partial(lax.fori_loop, 0, RT // WC, init_val=None)
    def _(r, _):
        z_copy(r).wait()
        return None

    plsc.subcore_barrier()

    def run_block(b, par):
        src_v, dst_v = idx[par]
        src_vo, dst_vo = idx[1 - par]

        @pl.when(b >= 1)
        def _():
            drain_scatter(dst_vo, IB - 1, NBUF - 1)

        @pl.when(b + 1 < NIB)
        def _():
            for d in stage_idx(b + 1, 1 - par):
                d.start()

        def step(j, p):
            vac = (p + NBUF - 1) % NBUF

            @pl.when(j >= 1)
            def _():
                drain_scatter(dst_v, j - 1, vac)

            @pl.when(jnp.logical_and(j == IB - NBUF, b + 1 < NIB))
            def _():
                for d in stage_idx(b + 1, 1 - par):
                    d.wait()

            @pl.when(j + NBUF - 1 < IB)
            def _():
                fire_gather(src_v, j + NBUF - 1, vac)

            @pl.when(jnp.logical_and(j + NBUF - 1 >= IB, b + 1 < NIB))
            def _():
                fire_gather(src_vo, j + NBUF - 1 - IB, vac)

            drain_gather(src_v, j, p)
            fire_scatter(dst_v, j, p)

        @functools.partial(lax.fori_loop, 0, IB, init_val=None)
        def _(j, _):
            for p in range(NBUF):
                @pl.when(j % NBUF == p)
                def _(p=p):
                    step(j, p)

            return None

    @functools.partial(lax.fori_loop, 0, NIB, init_val=None)
    def _(b, _):
        @pl.when(b % 2 == 0)
        def _():
            run_block(b, 0)

        @pl.when(b % 2 == 1)
        def _():
            run_block(b, 1)

        return None

    drain_scatter(idx[(NIB - 1) % 2][1], IB - 1, NBUF - 1)

    plsc.subcore_barrier()

    def wr_copy(r):
        return pltpu.make_async_copy(
            acc.at[pl.ds(tbase + r * WC, WC)],
            out_hbm.at[c].at[pl.ds(tbase + r * WC, WC)], gsem0)

    @functools.partial(lax.fori_loop, 0, RT // WC, init_val=None)
    def _(r, _):
        wr_copy(r).start()
        return None

    @functools.partial(lax.fori_loop, 0, RT // WC, init_val=None)
    def _(r, _):
        wr_copy(r).wait()
        return None


_sc_segment_sum = functools.partial(
    pl.kernel,
    out_type=jax.ShapeDtypeStruct((NC, NPAD, D), jnp.float32),
    mesh=plsc.VectorSubcoreMesh(
        core_axis_name="c", subcore_axis_name="s",
        num_cores=NC, num_subcores=NS),
    scratch_types=(
        [pltpu.VMEM((IB, C), jnp.int32)] * 4
        + [pltpu.VMEM((C, D), jnp.float32)] * NBUF
        + [pltpu.SemaphoreType.DMA] * (2 * NBUF + 1)
        + [pltpu.VMEM_SHARED((NPAD, D), jnp.float32)]
    ),
)(_sc_body)


BLK = 5000


def _mlp_body(relu_out, h_ref, p_ref, wa_ref, ba_ref, wb_ref, bb_ref, o_ref):
    z = h_ref[...] + p_ref[0] + p_ref[1]
    a = jnp.dot(z, wa_ref[...], preferred_element_type=jnp.float32) + ba_ref[...]
    a = jnp.where(a > 0, a, a * 0.01)
    o = jnp.dot(a, wb_ref[...], preferred_element_type=jnp.float32) + bb_ref[...]
    if relu_out:
        o = jnp.where(o > 0, o, o * 0.01)
    o_ref[...] = o


def _mlp_tc(h, p, wa_t, ba, wb_t, bb, relu_out):
    row_spec = pl.BlockSpec((BLK, D), lambda i: (i, 0))
    part_spec = pl.BlockSpec((2, BLK, D), lambda i: (0, i, 0))
    full_spec = pl.BlockSpec((D, D), lambda i: (0, 0))
    bias_spec = pl.BlockSpec((1, D), lambda i: (0, 0))
    return pl.pallas_call(
        functools.partial(_mlp_body, relu_out),
        grid=(N // BLK,),
        in_specs=[row_spec, part_spec,
                  full_spec, bias_spec, full_spec, bias_spec],
        out_specs=row_spec,
        out_shape=jax.ShapeDtypeStruct((N, D), jnp.float32),
    )(h, p, wa_t, ba.reshape(1, D), wb_t, bb.reshape(1, D))


def kernel(x, edge_index, W1a, b1a, W1b, b1b, W2a, b2a, W2b, b2b,
           W3a, b3a, W3b, b3b):
    src = edge_index[0].reshape(NW, NIB, IB, C)
    dst = edge_index[1].reshape(NW, NIB, IB, C)

    w3b_t = jnp.zeros((D, D), jnp.float32).at[:, :2].set(W3b.T)
    b3b_p = jnp.zeros((D,), jnp.float32).at[:2].set(b3b)

    p = _sc_segment_sum(x, src, dst)
    h = _mlp_tc(x, p, W1a.T, b1a, W1b.T, b1b, relu_out=True)

    p = _sc_segment_sum(h, src, dst)
    h = _mlp_tc(h, p, W2a.T, b2a, W2b.T, b2b, relu_out=True)

    p = _sc_segment_sum(h, src, dst)
    out = _mlp_tc(h, p, W3a.T, b3a, w3b_t, b3b_p, relu_out=False)

    return out[:, :2]
